# baseline (speedup 1.0000x reference)
import functools

import jax
import jax.numpy as jnp
from jax import lax
from jax.experimental import pallas as pl
from jax.experimental.pallas import tpu as pltpu
from jax.experimental.pallas import tpu_sc as plsc

_DIM = 32
_LAMBDA = 0.01
_CHUNK = 128
_SUPER = 128 // _DIM


def _sc_stage(h, r, t, ent_re, ent_im, rel_re, rel_im, n_rows):
    info = plsc.get_sparse_core_info()
    nc, ns = info.num_cores, info.num_subcores
    nw = nc * ns
    rows_per_w = n_rows // nw
    n_chunks = rows_per_w // _CHUNK
    n_groups = _CHUNK // 16
    mesh = plsc.VectorSubcoreMesh(core_axis_name="c", subcore_axis_name="s")

    @functools.partial(
        pl.kernel,
        mesh=mesh,
        compiler_params=pltpu.CompilerParams(needs_layout_passes=False),
        out_type=(
            jax.ShapeDtypeStruct((n_rows,), jnp.float32),
            jax.ShapeDtypeStruct((nw, 16), jnp.float32),
        ),
        scratch_types=[
            pltpu.VMEM((_CHUNK,), jnp.int32),
            pltpu.VMEM((_CHUNK,), jnp.int32),
            pltpu.VMEM((_CHUNK,), jnp.int32),
            pltpu.VMEM((_CHUNK,), jnp.int32),
            pltpu.VMEM((_CHUNK,), jnp.int32),
            pltpu.VMEM((_CHUNK,), jnp.int32),
            pltpu.VMEM((_CHUNK, 128), jnp.float32),
            pltpu.VMEM((_CHUNK, 128), jnp.float32),
            pltpu.VMEM((_CHUNK, 128), jnp.float32),
            pltpu.VMEM((_CHUNK, 128), jnp.float32),
            pltpu.VMEM((_CHUNK, 128), jnp.float32),
            pltpu.VMEM((_CHUNK, 128), jnp.float32),
            pltpu.VMEM((_CHUNK,), jnp.float32),
            pltpu.VMEM((16,), jnp.float32),
            pltpu.SemaphoreType.DMA,
        ],
    )
    def sc_kernel(h_hbm, r_hbm, t_hbm, ere_hbm, eim_hbm, rre_hbm, rim_hbm,
                  score_out, sq_out,
                  hi_v, ri_v, ti_v, hs_v, rs_v, ts_v,
                  beh, bih, bet, bit_, brr, bri,
                  sc_v, sq_v, sem):
        wid = lax.axis_index("s") * nc + lax.axis_index("c")
        base_w = wid * rows_per_w

        sq_v[...] = jnp.zeros((16,), jnp.float32)

        for g in range(n_chunks):
            base = base_w + g * _CHUNK
            pltpu.sync_copy(h_hbm.at[pl.ds(base, _CHUNK)], hi_v)
            pltpu.sync_copy(r_hbm.at[pl.ds(base, _CHUNK)], ri_v)
            pltpu.sync_copy(t_hbm.at[pl.ds(base, _CHUNK)], ti_v)
            for g2 in range(n_groups):
                sl = pl.ds(g2 * 16, 16)
                hs_v[sl] = lax.shift_right_logical(hi_v[sl], 2)
                rs_v[sl] = lax.shift_right_logical(ri_v[sl], 2)
                ts_v[sl] = lax.shift_right_logical(ti_v[sl], 2)
            d0 = pltpu.async_copy(ere_hbm.at[hs_v], beh, sem)
            d1 = pltpu.async_copy(eim_hbm.at[hs_v], bih, sem)
            d2 = pltpu.async_copy(ere_hbm.at[ts_v], bet, sem)
            d3 = pltpu.async_copy(eim_hbm.at[ts_v], bit_, sem)
            d4 = pltpu.async_copy(rre_hbm.at[rs_v], brr, sem)
            d5 = pltpu.async_copy(rim_hbm.at[rs_v], bri, sem)
            d0.wait(); d1.wait(); d2.wait(); d3.wait(); d4.wait(); d5.wait()

            def group(g2, sq):
                sl = pl.ds(g2 * 16, 16)
                j = lax.iota(jnp.int32, 16) + g2 * 16
                offh = (hi_v[sl] & (_SUPER - 1)) * _DIM
                offt = (ti_v[sl] & (_SUPER - 1)) * _DIM
                offr = (ri_v[sl] & (_SUPER - 1)) * _DIM
                score = jnp.zeros((16,), jnp.float32)
                for d in range(_DIM):
                    ch = offh + d
                    ct = offt + d
                    cr = offr + d
                    reh = plsc.load_gather(beh, [j, ch])
                    imh = plsc.load_gather(bih, [j, ch])
                    ret_ = plsc.load_gather(bet, [j, ct])
                    imt = plsc.load_gather(bit_, [j, ct])
                    rre = plsc.load_gather(brr, [j, cr])
                    rim = plsc.load_gather(bri, [j, cr])
                    score = score + (rre * (reh * ret_ + imh * imt)
                                     + rim * (reh * imt - imh * ret_))
                    sq = sq + (reh * reh + imh * imh + ret_ * ret_
                               + imt * imt + rre * rre + rim * rim)
                sc_v[sl] = score
                return sq

            sq = lax.fori_loop(0, n_groups, group, sq_v[...])
            sq_v[...] = sq
            pltpu.sync_copy(sc_v, score_out.at[pl.ds(base, _CHUNK)])

        pltpu.sync_copy(sq_v, sq_out.at[wid])

    return sc_kernel(h, r, t, ent_re, ent_im, rel_re, rel_im)


def _tc_reduce(score, sq, n_rows):
    def body(s_ref, sq_ref, o_ref):
        s = s_ref[...]
        sp = jnp.maximum(s, 0.0) + jnp.log(1.0 + jnp.exp(-jnp.abs(s)))
        loss = jnp.sum(sp) * (1.0 / n_rows)
        regul = jnp.sum(sq_ref[...]) * (1.0 / (n_rows * _DIM))
        o_ref[0, 0] = loss + _LAMBDA * regul

    out = pl.pallas_call(
        body,
        out_shape=jax.ShapeDtypeStruct((1, 1), jnp.float32),
        out_specs=pl.BlockSpec(memory_space=pltpu.SMEM),
    )(score.reshape(n_rows // 128, 128), sq)
    return out[0, 0]


def kernel(pos_h, pos_r, pos_t, neg_h, neg_r, neg_t, ent_re, ent_im, rel_re, rel_im):
    h = jnp.concatenate([pos_h, neg_h])
    r = jnp.concatenate([pos_r, neg_r])
    t = jnp.concatenate([pos_t, neg_t])
    n_rows = h.shape[0]
    ere = ent_re.reshape(-1, 128)
    eim = ent_im.reshape(-1, 128)
    rre = rel_re.reshape(-1, 128)
    rim = rel_im.reshape(-1, 128)
    score, sq = _sc_stage(h, r, t, ere, eim, rre, rim, n_rows)
    return _tc_reduce(score, sq, n_rows)


# speedup vs baseline: 2.0204x; 2.0204x over previous
import functools

import jax
import jax.numpy as jnp
from jax import lax
from jax.experimental import pallas as pl
from jax.experimental.pallas import tpu as pltpu
from jax.experimental.pallas import tpu_sc as plsc

_DIM = 32
_LAMBDA = 0.01
_CHUNK = 128
_EB = 2048


def _pack_tables(ereT, eimT, n_ent):
    n_blocks = (n_ent + _EB - 1) // _EB
    n_sub = _EB // 256

    def body(a_ref, b_ref, o_ref):
        for k in range(n_sub):
            s0 = pl.ds(256 * k, 128)
            s1 = pl.ds(256 * k + 128, 128)
            stacked = jnp.concatenate(
                [a_ref[:, s0], b_ref[:, s0], a_ref[:, s1], b_ref[:, s1]],
                axis=0)
            o_ref[pl.ds(128 * k, 128), :] = stacked.T

    return pl.pallas_call(
        body,
        grid=(n_blocks,),
        in_specs=[
            pl.BlockSpec((_DIM, _EB), lambda g: (0, g)),
            pl.BlockSpec((_DIM, _EB), lambda g: (0, g)),
        ],
        out_specs=pl.BlockSpec((_EB // 2, 128), lambda g: (g, 0)),
        out_shape=jax.ShapeDtypeStruct((n_blocks * (_EB // 2), 128), jnp.float32),
    )(ereT, eimT)


def _sc_stage(h, r, t, packed, rreT, rimT, n_rows, n_rel):
    info = plsc.get_sparse_core_info()
    nc, ns = info.num_cores, info.num_subcores
    nw = nc * ns
    rows_per_w = n_rows // nw
    n_chunks = rows_per_w // _CHUNK
    n_groups = _CHUNK // 16
    mesh = plsc.VectorSubcoreMesh(core_axis_name="c", subcore_axis_name="s")

    @functools.partial(
        pl.kernel,
        mesh=mesh,
        compiler_params=pltpu.CompilerParams(needs_layout_passes=False),
        out_type=(
            jax.ShapeDtypeStruct((n_rows,), jnp.float32),
            jax.ShapeDtypeStruct((nw, 16), jnp.float32),
        ),
        scratch_types=[
            pltpu.VMEM((_CHUNK,), jnp.int32),
            pltpu.VMEM((_CHUNK,), jnp.int32),
            pltpu.VMEM((_CHUNK,), jnp.int32),
            pltpu.VMEM((_CHUNK,), jnp.int32),
            pltpu.VMEM((_CHUNK,), jnp.int32),
            pltpu.VMEM((_CHUNK, 128), jnp.float32),
            pltpu.VMEM((_CHUNK, 128), jnp.float32),
            pltpu.VMEM((_DIM, n_rel), jnp.float32),
            pltpu.VMEM((_DIM, n_rel), jnp.float32),
            pltpu.VMEM((_CHUNK,), jnp.float32),
            pltpu.VMEM((16,), jnp.float32),
            pltpu.SemaphoreType.DMA,
            pltpu.SemaphoreType.DMA,
        ],
    )
    def sc_kernel(h_hbm, r_hbm, t_hbm, packed_hbm, rreT_hbm, rimT_hbm,
                  score_out, sq_out,
                  hi_v, ri_v, ti_v, qh_v, qt_v, bh, bt, vrr, vri,
                  sc_v, sq_v, sem, sem2):
        wid = lax.axis_index("s") * nc + lax.axis_index("c")
        base_w = wid * rows_per_w

        rd0 = pltpu.async_copy(rreT_hbm, vrr, sem2)
        rd1 = pltpu.async_copy(rimT_hbm, vri, sem2)
        sq_v[...] = jnp.zeros((16,), jnp.float32)
        rd0.wait()
        rd1.wait()

        for g in range(n_chunks):
            base = base_w + g * _CHUNK
            pltpu.sync_copy(h_hbm.at[pl.ds(base, _CHUNK)], hi_v)
            pltpu.sync_copy(r_hbm.at[pl.ds(base, _CHUNK)], ri_v)
            pltpu.sync_copy(t_hbm.at[pl.ds(base, _CHUNK)], ti_v)
            for g2 in range(n_groups):
                sl = pl.ds(g2 * 16, 16)
                e = hi_v[sl]
                qh_v[sl] = lax.shift_left(lax.shift_right_logical(e, 8), 7) | (e & 127)
                e = ti_v[sl]
                qt_v[sl] = lax.shift_left(lax.shift_right_logical(e, 8), 7) | (e & 127)
            d0 = pltpu.async_copy(packed_hbm.at[qh_v], bh, sem)
            d1 = pltpu.async_copy(packed_hbm.at[qt_v], bt, sem)
            d0.wait()
            d1.wait()

            def group(g2, sq):
                sl = pl.ds(g2 * 16, 16)
                j16 = lax.iota(jnp.int32, 16) + g2 * 16
                offh = (lax.shift_right_logical(hi_v[sl], 7) & 1) * 64
                offt = (lax.shift_right_logical(ti_v[sl], 7) & 1) * 64
                r16 = ri_v[sl]
                score = jnp.zeros((16,), jnp.float32)
                for c in range(_DIM):
                    cc = jnp.full((16,), c, jnp.int32)
                    reh = plsc.load_gather(bh, [j16, offh + c])
                    imh = plsc.load_gather(bh, [j16, offh + (32 + c)])
                    ret_ = plsc.load_gather(bt, [j16, offt + c])
                    imt = plsc.load_gather(bt, [j16, offt + (32 + c)])
                    rre = plsc.load_gather(vrr, [cc, r16])
                    rim = plsc.load_gather(vri, [cc, r16])
                    score = score + (rre * (reh * ret_ + imh * imt)
                                     + rim * (reh * imt - imh * ret_))
                    sq = sq + (reh * reh + imh * imh + ret_ * ret_
                               + imt * imt + rre * rre + rim * rim)
                sc_v[sl] = score
                return sq

            sq = lax.fori_loop(0, n_groups, group, sq_v[...])
            sq_v[...] = sq
            pltpu.sync_copy(sc_v, score_out.at[pl.ds(base, _CHUNK)])

        pltpu.sync_copy(sq_v, sq_out.at[wid])

    return sc_kernel(h, r, t, packed, rreT, rimT)


def _tc_reduce(score, sq, n_rows):
    def body(s_ref, sq_ref, o_ref):
        s = s_ref[...]
        sp = jnp.maximum(s, 0.0) + jnp.log(1.0 + jnp.exp(-jnp.abs(s)))
        loss = jnp.sum(sp) * (1.0 / n_rows)
        regul = jnp.sum(sq_ref[...]) * (1.0 / (n_rows * _DIM))
        o_ref[0, 0] = loss + _LAMBDA * regul

    out = pl.pallas_call(
        body,
        out_shape=jax.ShapeDtypeStruct((1, 1), jnp.float32),
        out_specs=pl.BlockSpec(memory_space=pltpu.SMEM),
    )(score.reshape(n_rows // 128, 128), sq)
    return out[0, 0]


def kernel(pos_h, pos_r, pos_t, neg_h, neg_r, neg_t, ent_re, ent_im, rel_re, rel_im):
    h = jnp.concatenate([pos_h, neg_h])
    r = jnp.concatenate([pos_r, neg_r])
    t = jnp.concatenate([pos_t, neg_t])
    n_rows = h.shape[0]
    packed = _pack_tables(ent_re.T, ent_im.T, ent_re.shape[0])
    score, sq = _sc_stage(h, r, t, packed, rel_re.T, rel_im.T,
                          n_rows, rel_re.shape[0])
    return _tc_reduce(score, sq, n_rows)


# speedup vs baseline: 2.7538x; 1.3630x over previous
import functools

import jax
import jax.numpy as jnp
from jax import lax
from jax.experimental import pallas as pl
from jax.experimental.pallas import tpu as pltpu
from jax.experimental.pallas import tpu_sc as plsc

_DIM = 32
_LAMBDA = 0.01
_CHUNK = 64
_EB = 4096


def _pack_tables(ereT, eimT, n_ent):
    n_blocks = (n_ent + _EB - 1) // _EB
    n_sub = _EB // 256

    def body(a_ref, b_ref, o_ref):
        for k in range(n_sub):
            s0 = pl.ds(256 * k, 128)
            s1 = pl.ds(256 * k + 128, 128)
            stacked = jnp.concatenate(
                [a_ref[:, s0], b_ref[:, s0], a_ref[:, s1], b_ref[:, s1]],
                axis=0)
            o_ref[pl.ds(128 * k, 128), :] = stacked.T

    return pl.pallas_call(
        body,
        grid=(n_blocks,),
        in_specs=[
            pl.BlockSpec((_DIM, _EB), lambda g: (0, g)),
            pl.BlockSpec((_DIM, _EB), lambda g: (0, g)),
        ],
        out_specs=pl.BlockSpec((_EB // 2, 128), lambda g: (g, 0)),
        out_shape=jax.ShapeDtypeStruct((n_blocks * (_EB // 2), 128), jnp.float32),
    )(ereT, eimT)


def _sc_stage(h, r, t, packed, rreT, rimT, n_rows, n_rel):
    info = plsc.get_sparse_core_info()
    nc, ns = info.num_cores, info.num_subcores
    nw = nc * ns
    rows_per_w = n_rows // nw
    n_chunks = rows_per_w // _CHUNK
    n_groups = _CHUNK // 16
    mesh = plsc.VectorSubcoreMesh(core_axis_name="c", subcore_axis_name="s")

    @functools.partial(
        pl.kernel,
        mesh=mesh,
        compiler_params=pltpu.CompilerParams(needs_layout_passes=False),
        out_type=(
            jax.ShapeDtypeStruct((n_rows,), jnp.float32),
            jax.ShapeDtypeStruct((nw, 16), jnp.float32),
        ),
        scratch_types=[
            pltpu.VMEM((2, _CHUNK), jnp.int32),
            pltpu.VMEM((2, _CHUNK), jnp.int32),
            pltpu.VMEM((2, _CHUNK), jnp.int32),
            pltpu.VMEM((2, _CHUNK), jnp.int32),
            pltpu.VMEM((2, _CHUNK), jnp.int32),
            pltpu.VMEM((2, _CHUNK, 128), jnp.float32),
            pltpu.VMEM((2, _CHUNK, 128), jnp.float32),
            pltpu.VMEM((_DIM, n_rel), jnp.float32),
            pltpu.VMEM((_DIM, n_rel), jnp.float32),
            pltpu.VMEM((2, _CHUNK), jnp.float32),
            pltpu.VMEM((16,), jnp.float32),
            pltpu.SemaphoreType.DMA,
            pltpu.SemaphoreType.DMA,
            pltpu.SemaphoreType.DMA,
        ],
    )
    def sc_kernel(h_hbm, r_hbm, t_hbm, packed_hbm, rreT_hbm, rimT_hbm,
                  score_out, sq_out,
                  hi_v, ri_v, ti_v, qh_v, qt_v, bh, bt, vrr, vri,
                  sc_v, sq_v, sem_i, sem_g, sem_o):
        wid = lax.axis_index("s") * nc + lax.axis_index("c")
        base_w = wid * rows_per_w

        rd0 = pltpu.async_copy(rreT_hbm, vrr, sem_o)
        rd1 = pltpu.async_copy(rimT_hbm, vri, sem_o)
        sq_v[...] = jnp.zeros((16,), jnp.float32)

        def load_idx(g, b):
            base = base_w + g * _CHUNK
            return (
                pltpu.async_copy(h_hbm.at[pl.ds(base, _CHUNK)], hi_v.at[b], sem_i),
                pltpu.async_copy(r_hbm.at[pl.ds(base, _CHUNK)], ri_v.at[b], sem_i),
                pltpu.async_copy(t_hbm.at[pl.ds(base, _CHUNK)], ti_v.at[b], sem_i),
            )

        def start_gather(b):
            for g2 in range(n_groups):
                sl = pl.ds(g2 * 16, 16)
                e = hi_v[b, sl]
                qh_v[b, sl] = lax.shift_left(lax.shift_right_logical(e, 8), 7) | (e & 127)
                e = ti_v[b, sl]
                qt_v[b, sl] = lax.shift_left(lax.shift_right_logical(e, 8), 7) | (e & 127)
            return (
                pltpu.async_copy(packed_hbm.at[qh_v.at[b]], bh.at[b], sem_g),
                pltpu.async_copy(packed_hbm.at[qt_v.at[b]], bt.at[b], sem_g),
            )

        def compute(g, b):
            def group(g2, sq):
                sl = pl.ds(g2 * 16, 16)
                j16 = lax.iota(jnp.int32, 16) + g2 * 16
                offh = (lax.shift_right_logical(hi_v[b, sl], 7) & 1) * 64
                offt = (lax.shift_right_logical(ti_v[b, sl], 7) & 1) * 64
                r16 = ri_v[b, sl]
                score = jnp.zeros((16,), jnp.float32)
                for c in range(_DIM):
                    cc = jnp.full((16,), c, jnp.int32)
                    reh = plsc.load_gather(bh.at[b], [j16, offh + c])
                    imh = plsc.load_gather(bh.at[b], [j16, offh + (32 + c)])
                    ret_ = plsc.load_gather(bt.at[b], [j16, offt + c])
                    imt = plsc.load_gather(bt.at[b], [j16, offt + (32 + c)])
                    rre = plsc.load_gather(vrr, [cc, r16])
                    rim = plsc.load_gather(vri, [cc, r16])
                    score = score + (rre * (reh * ret_ + imh * imt)
                                     + rim * (reh * imt - imh * ret_))
                    sq = sq + (reh * reh + imh * imh + ret_ * ret_
                               + imt * imt + rre * rre + rim * rim)
                sc_v[b, sl] = score
                return sq

            sq = lax.fori_loop(0, n_groups, group, sq_v[...])
            sq_v[...] = sq
            base = base_w + g * _CHUNK
            return pltpu.async_copy(
                sc_v.at[b], score_out.at[pl.ds(base, _CHUNK)], sem_o)

        i0 = load_idx(0, 0)
        rd0.wait()
        rd1.wait()
        for d in i0:
            d.wait()
        g_prev = start_gather(0)
        out_prev = None
        for g in range(n_chunks):
            b = g & 1
            if g + 1 < n_chunks:
                i1 = load_idx(g + 1, b ^ 1)
            g_prev[0].wait()
            g_prev[1].wait()
            if g + 1 < n_chunks:
                for d in i1:
                    d.wait()
                g_next = start_gather(b ^ 1)
            if out_prev is not None:
                out_prev.wait()
            out_prev = compute(g, b)
            if g + 1 < n_chunks:
                g_prev = g_next
        out_prev.wait()

        pltpu.sync_copy(sq_v, sq_out.at[wid])

    return sc_kernel(h, r, t, packed, rreT, rimT)


def _tc_reduce(score, sq, n_rows):
    def body(s_ref, sq_ref, o_ref):
        s = s_ref[...]
        sp = jnp.maximum(s, 0.0) + jnp.log(1.0 + jnp.exp(-jnp.abs(s)))
        loss = jnp.sum(sp) * (1.0 / n_rows)
        regul = jnp.sum(sq_ref[...]) * (1.0 / (n_rows * _DIM))
        o_ref[0, 0] = loss + _LAMBDA * regul

    out = pl.pallas_call(
        body,
        out_shape=jax.ShapeDtypeStruct((1, 1), jnp.float32),
        out_specs=pl.BlockSpec(memory_space=pltpu.SMEM),
    )(score.reshape(n_rows // 128, 128), sq)
    return out[0, 0]


def kernel(pos_h, pos_r, pos_t, neg_h, neg_r, neg_t, ent_re, ent_im, rel_re, rel_im):
    h = jnp.concatenate([pos_h, neg_h])
    r = jnp.concatenate([pos_r, neg_r])
    t = jnp.concatenate([pos_t, neg_t])
    n_rows = h.shape[0]
    packed = _pack_tables(ent_re.T, ent_im.T, ent_re.shape[0])
    score, sq = _sc_stage(h, r, t, packed, rel_re.T, rel_im.T,
                          n_rows, rel_re.shape[0])
    return _tc_reduce(score, sq, n_rows)
